# baseline (speedup 1.0000x reference)
import functools

import jax
import jax.numpy as jnp
from jax import lax
from jax.experimental import pallas as pl
from jax.experimental.pallas import tpu as pltpu
from jax.experimental.pallas import tpu_sc as plsc

N = 10000
NP = 10240
E_EDGES = 160000
EP = 163840
D_EDGE = 16
FG = 128
F1 = 112
F2 = 48
NC, NS = 2, 16
NW = NC * NS
CH = 40
PT = EP // NW
MBLK = 1000
GBLK = 2048


def _lin_body(h_ref, w_ref, b_ref, o_ref):
    o_ref[...] = jnp.dot(h_ref[...], w_ref[...],
                         preferred_element_type=jnp.float32) + b_ref[...]


def _node_linear(h, w, b):
    m, k = h.shape
    f = w.shape[1]
    return pl.pallas_call(
        _lin_body,
        grid=(m // MBLK,),
        in_specs=[
            pl.BlockSpec((MBLK, k), lambda i: (i, 0)),
            pl.BlockSpec((k, f), lambda i: (0, 0)),
            pl.BlockSpec((1, f), lambda i: (0, 0)),
        ],
        out_specs=pl.BlockSpec((MBLK, f), lambda i: (i, 0)),
        out_shape=jax.ShapeDtypeStruct((m, f), jnp.float32),
    )(h, w, b)


def _gate_body(e_ref, w_ref, b_ref, o_ref):
    x = jnp.dot(e_ref[...], w_ref[...],
                preferred_element_type=jnp.float32) + b_ref[...]
    o_ref[...] = jax.nn.sigmoid(x)


def _edge_gate(e, w, b):
    f = w.shape[1]
    last_blk = E_EDGES // GBLK
    return pl.pallas_call(
        _gate_body,
        grid=(EP // GBLK,),
        in_specs=[
            pl.BlockSpec((GBLK, D_EDGE), lambda i: (jnp.minimum(i, last_blk), 0)),
            pl.BlockSpec((D_EDGE, f), lambda i: (0, 0)),
            pl.BlockSpec((1, f), lambda i: (0, 0)),
        ],
        out_specs=pl.BlockSpec((GBLK, f), lambda i: (i, 0)),
        out_shape=jax.ShapeDtypeStruct((EP, f), jnp.float32),
    )(e, w, b)


def _layer2_body(p0_ref, p1_ref, l_ref, w_ref, b_ref, o_ref):
    h = jnp.maximum(p0_ref[...] + p1_ref[...] + l_ref[...], 0.0)
    o_ref[...] = jnp.dot(h, w_ref[...],
                         preferred_element_type=jnp.float32) + b_ref[...]


def _layer2_linear(p0, p1, lin1, w, b):
    f = w.shape[1]
    return pl.pallas_call(
        _layer2_body,
        grid=(N // MBLK,),
        in_specs=[
            pl.BlockSpec((MBLK, FG), lambda i: (i, 0)),
            pl.BlockSpec((MBLK, FG), lambda i: (i, 0)),
            pl.BlockSpec((MBLK, FG), lambda i: (i, 0)),
            pl.BlockSpec((FG, f), lambda i: (0, 0)),
            pl.BlockSpec((1, f), lambda i: (0, 0)),
        ],
        out_specs=pl.BlockSpec((MBLK, f), lambda i: (i, 0)),
        out_shape=jax.ShapeDtypeStruct((N, f), jnp.float32),
    )(p0, p1, lin1, w, b)


def _final_body(p0_ref, p1_ref, l_ref, o_ref):
    x = p0_ref[...] + p1_ref[...] + l_ref[...]
    col = lax.broadcasted_iota(jnp.int32, x.shape, 1)
    xm = jnp.where(col < 40, x, -1e30)
    m = jnp.max(xm, axis=1, keepdims=True)
    lse = jnp.log(jnp.sum(jnp.exp(xm - m), axis=1, keepdims=True)) + m
    o_ref[...] = (x - lse)[:, :40]


def _final_logsoftmax(p0, p1, lin2):
    return pl.pallas_call(
        _final_body,
        grid=(N // MBLK,),
        in_specs=[
            pl.BlockSpec((MBLK, FG), lambda i: (i, 0)),
            pl.BlockSpec((MBLK, FG), lambda i: (i, 0)),
            pl.BlockSpec((MBLK, FG), lambda i: (i, 0)),
        ],
        out_specs=pl.BlockSpec((MBLK, 40), lambda i: (i, 0)),
        out_shape=jax.ShapeDtypeStruct((N, 40), jnp.float32),
    )(p0, p1, lin2)


def _sc_gather_mul_scatter(lin, gate, src2, dst2, fv):
    rows_per_sub = NP // NS
    n_ch = PT // CH
    mesh = plsc.VectorSubcoreMesh(core_axis_name="c", subcore_axis_name="s")

    @functools.partial(
        pl.kernel,
        out_type=jax.ShapeDtypeStruct((NC, NP, FG), jnp.float32),
        mesh=mesh,
        scratch_types=[
            pltpu.VMEM_SHARED((NP, FG), jnp.float32),
            pltpu.VMEM((n_ch, CH), jnp.int32),
            pltpu.VMEM((1, CH), jnp.int32),
            pltpu.VMEM((1, CH), jnp.int32),
            pltpu.VMEM((CH, FG), jnp.float32),
            pltpu.VMEM((CH, FG), jnp.float32),
            pltpu.VMEM((CH, FG), jnp.float32),
            pltpu.VMEM((CH, FG), jnp.float32),
            pltpu.SemaphoreType.DMA,
            pltpu.SemaphoreType.DMA,
            pltpu.SemaphoreType.DMA,
            pltpu.SemaphoreType.DMA,
            pltpu.SemaphoreType.DMA,
            pltpu.SemaphoreType.DMA,
            pltpu.SemaphoreType.DMA,
            pltpu.SemaphoreType.DMA,
        ],
    )
    def sc_kernel(lin_hbm, gate_hbm, src_hbm, dst_hbm, out_hbm,
                  acc, src_all, dst0, dst1, rows0, rows1, gv0, gv1,
                  sem_r0, sem_r1, sem_g0, sem_g1,
                  sem_d0, sem_d1, sem_s0, sem_s1):
        cid = lax.axis_index("c")
        sid = lax.axis_index("s")
        wid = sid * NC + cid
        cb = wid * n_ch

        bufs = ((rows0, gv0, dst0, sem_r0, sem_g0, sem_d0, sem_s0),
                (rows1, gv1, dst1, sem_r1, sem_g1, sem_d1, sem_s1))

        @pl.loop(0, CH)
        def _(i):
            @pl.loop(0, FG, step=16)
            def _(q):
                rows0[i, pl.ds(q, 16)] = jnp.zeros((16,), jnp.float32)

        @pl.loop(0, rows_per_sub, step=CH)
        def _(r):
            pltpu.sync_copy(rows0, acc.at[pl.ds(sid * rows_per_sub + r, CH)])

        pltpu.sync_copy(src_hbm.at[pl.ds(cb, n_ch)], src_all)

        plsc.subcore_barrier()

        def fire(cg, rows, gv, dstv, sem_r, sem_g, sem_d):
            pltpu.async_copy(lin_hbm.at[src_all.at[cg]], rows, sem_r)
            pltpu.async_copy(gate_hbm.at[pl.ds((cb + cg) * CH, CH)], gv, sem_g)
            pltpu.async_copy(dst_hbm.at[pl.ds(cb + cg, 1)], dstv, sem_d)

        for b in range(2):
            rows, gv, dstv, sem_r, sem_g, sem_d, _ = bufs[b]
            fire(b, rows, gv, dstv, sem_r, sem_g, sem_d)

        @pl.loop(0, n_ch, step=2)
        def _(c):
            for b in range(2):
                rows, gv, dstv, sem_r, sem_g, sem_d, sem_s = bufs[b]
                cg = c + b
                pltpu.make_async_copy(lin_hbm.at[src_all.at[cg]], rows,
                                      sem_r).wait()
                pltpu.make_async_copy(gate_hbm.at[pl.ds((cb + cg) * CH, CH)],
                                      gv, sem_g).wait()
                pltpu.make_async_copy(dst_hbm.at[pl.ds(cb + cg, 1)], dstv,
                                      sem_d).wait()

                @pl.loop(0, fv, step=16)
                def _(q, rows=rows, gv=gv):
                    @pl.loop(0, CH, step=4)
                    def _(i, q=q, rows=rows, gv=gv):
                        for u in range(4):
                            rows[i + u, pl.ds(q, 16)] = (
                                rows[i + u, pl.ds(q, 16)]
                                * gv[i + u, pl.ds(q, 16)])


            for b in range(2):
                rows, gv, dstv, sem_r, sem_g, sem_d, sem_s = bufs[b]
                cg = c + b

                @pl.when(cg + 2 < n_ch)
                def _(rows=rows, gv=gv, dstv=dstv, cg=cg,
                      sem_r=sem_r, sem_g=sem_g, sem_d=sem_d, sem_s=sem_s):
                    fire(cg + 2, rows, gv, dstv, sem_r, sem_g, sem_d)


        plsc.subcore_barrier()

        @pl.loop(0, rows_per_sub, step=CH)
        def _(r):
            row = sid * rows_per_sub + r
            pltpu.sync_copy(acc.at[pl.ds(row, CH)],
                            out_hbm.at[cid, pl.ds(row, CH)])

    return sc_kernel(lin, gate, src2, dst2)


def kernel(H, A, E, W1, b1, We1, be1, W2, b2, We2, be2):
    pad_e = EP - E_EDGES
    src = A[0]
    dst = A[1]
    srcp = jnp.concatenate([src, jnp.zeros((pad_e,), jnp.int32)])
    dstp = jnp.concatenate(
        [dst, N + (jnp.arange(pad_e, dtype=jnp.int32) % (NP - N))])
    src2 = srcp.reshape(EP // CH, CH)
    dst2 = dstp.reshape(EP // CH, CH)

    W1p = jnp.pad(W1, ((0, 0), (0, FG - 100)))
    b1p = jnp.pad(b1, (0, FG - 100)).reshape(1, FG)
    We1p = jnp.pad(We1, ((0, 0), (0, FG - 100)))
    be1p = jnp.pad(be1, (0, FG - 100)).reshape(1, FG)
    W2p = jnp.pad(W2, ((0, FG - 100), (0, FG - 40)))
    b2p = jnp.pad(b2, (0, FG - 40)).reshape(1, FG)
    We2p = jnp.pad(We2, ((0, 0), (0, FG - 40)))
    be2p = jnp.pad(be2, (0, FG - 40)).reshape(1, FG)

    lin1 = _node_linear(H, W1p, b1p)
    gate1 = _edge_gate(E, We1p, be1p)
    p1 = _sc_gather_mul_scatter(lin1, gate1, src2, dst2, F1)
    gate2 = _edge_gate(E, We2p, be2p)
    lin2 = _layer2_linear(p1[0], p1[1], lin1, W2p, b2p)
    p2 = _sc_gather_mul_scatter(lin2, gate2, src2, dst2, F2)
    return _final_logsoftmax(p2[0], p2[1], lin2)


# speedup vs baseline: 1.0977x; 1.0977x over previous
import functools

import jax
import jax.numpy as jnp
from jax import lax
from jax.experimental import pallas as pl
from jax.experimental.pallas import tpu as pltpu
from jax.experimental.pallas import tpu_sc as plsc

N = 10000
NP = 10240
E_EDGES = 160000
EP = 163840
D_EDGE = 16
FG = 128
F1 = 112
F2 = 48
NC, NS = 2, 16
NW = NC * NS
CH = 40
PT = EP // NW
MBLK = 1000
GBLK = 2048


def _lin_body(h_ref, w_ref, b_ref, o_ref):
    o_ref[...] = jnp.dot(h_ref[...], w_ref[...],
                         preferred_element_type=jnp.float32) + b_ref[...]


def _node_linear(h, w, b):
    m, k = h.shape
    f = w.shape[1]
    return pl.pallas_call(
        _lin_body,
        grid=(m // MBLK,),
        in_specs=[
            pl.BlockSpec((MBLK, k), lambda i: (i, 0)),
            pl.BlockSpec((k, f), lambda i: (0, 0)),
            pl.BlockSpec((1, f), lambda i: (0, 0)),
        ],
        out_specs=pl.BlockSpec((MBLK, f), lambda i: (i, 0)),
        out_shape=jax.ShapeDtypeStruct((m, f), jnp.float32),
    )(h, w, b)


def _gate_body(e_ref, w_ref, b_ref, o_ref):
    x = jnp.dot(e_ref[...].astype(jnp.bfloat16),
                w_ref[...].astype(jnp.bfloat16),
                preferred_element_type=jnp.float32) + b_ref[...]
    o_ref[...] = jax.nn.sigmoid(x)


def _edge_gate(e, w, b):
    f = w.shape[1]
    last_blk = E_EDGES // GBLK
    return pl.pallas_call(
        _gate_body,
        grid=(EP // GBLK,),
        in_specs=[
            pl.BlockSpec((GBLK, D_EDGE), lambda i: (jnp.minimum(i, last_blk), 0)),
            pl.BlockSpec((D_EDGE, f), lambda i: (0, 0)),
            pl.BlockSpec((1, f), lambda i: (0, 0)),
        ],
        out_specs=pl.BlockSpec((GBLK, f), lambda i: (i, 0)),
        out_shape=jax.ShapeDtypeStruct((EP, f), jnp.float32),
    )(e, w, b)


def _layer2_body(p0_ref, p1_ref, l_ref, w_ref, b_ref, o_ref):
    h = jnp.maximum(p0_ref[...] + p1_ref[...] + l_ref[...], 0.0)
    o_ref[...] = jnp.dot(h, w_ref[...],
                         preferred_element_type=jnp.float32) + b_ref[...]


def _layer2_linear(p0, p1, lin1, w, b):
    f = w.shape[1]
    return pl.pallas_call(
        _layer2_body,
        grid=(N // MBLK,),
        in_specs=[
            pl.BlockSpec((MBLK, FG), lambda i: (i, 0)),
            pl.BlockSpec((MBLK, FG), lambda i: (i, 0)),
            pl.BlockSpec((MBLK, FG), lambda i: (i, 0)),
            pl.BlockSpec((FG, f), lambda i: (0, 0)),
            pl.BlockSpec((1, f), lambda i: (0, 0)),
        ],
        out_specs=pl.BlockSpec((MBLK, f), lambda i: (i, 0)),
        out_shape=jax.ShapeDtypeStruct((N, f), jnp.float32),
    )(p0, p1, lin1, w, b)


def _final_body(p0_ref, p1_ref, l_ref, o_ref):
    x = p0_ref[...] + p1_ref[...] + l_ref[...]
    col = lax.broadcasted_iota(jnp.int32, x.shape, 1)
    xm = jnp.where(col < 40, x, -1e30)
    m = jnp.max(xm, axis=1, keepdims=True)
    lse = jnp.log(jnp.sum(jnp.exp(xm - m), axis=1, keepdims=True)) + m
    o_ref[...] = (x - lse)[:, :40]


def _final_logsoftmax(p0, p1, lin2):
    return pl.pallas_call(
        _final_body,
        grid=(N // MBLK,),
        in_specs=[
            pl.BlockSpec((MBLK, FG), lambda i: (i, 0)),
            pl.BlockSpec((MBLK, FG), lambda i: (i, 0)),
            pl.BlockSpec((MBLK, FG), lambda i: (i, 0)),
        ],
        out_specs=pl.BlockSpec((MBLK, 40), lambda i: (i, 0)),
        out_shape=jax.ShapeDtypeStruct((N, 40), jnp.float32),
    )(p0, p1, lin2)


def _sc_gather_mul_scatter(lin, gate, src2, dst2, fv):
    rows_per_sub = NP // NS
    n_ch = PT // CH
    mesh = plsc.VectorSubcoreMesh(core_axis_name="c", subcore_axis_name="s")

    @functools.partial(
        pl.kernel,
        out_type=jax.ShapeDtypeStruct((NC, NP, FG), jnp.float32),
        mesh=mesh,
        scratch_types=[
            pltpu.VMEM_SHARED((NP, FG), jnp.float32),
            pltpu.VMEM((n_ch, CH), jnp.int32),
            pltpu.VMEM((1, CH), jnp.int32),
            pltpu.VMEM((1, CH), jnp.int32),
            pltpu.VMEM((CH, FG), jnp.float32),
            pltpu.VMEM((CH, FG), jnp.float32),
            pltpu.VMEM((CH, FG), jnp.float32),
            pltpu.VMEM((CH, FG), jnp.float32),
            pltpu.SemaphoreType.DMA,
            pltpu.SemaphoreType.DMA,
            pltpu.SemaphoreType.DMA,
            pltpu.SemaphoreType.DMA,
            pltpu.SemaphoreType.DMA,
            pltpu.SemaphoreType.DMA,
            pltpu.SemaphoreType.DMA,
            pltpu.SemaphoreType.DMA,
        ],
    )
    def sc_kernel(lin_hbm, gate_hbm, src_hbm, dst_hbm, out_hbm,
                  acc, src_all, dst0, dst1, rows0, rows1, gv0, gv1,
                  sem_r0, sem_r1, sem_g0, sem_g1,
                  sem_d0, sem_d1, sem_s0, sem_s1):
        cid = lax.axis_index("c")
        sid = lax.axis_index("s")
        wid = sid * NC + cid
        cb = wid * n_ch

        bufs = ((rows0, gv0, dst0, sem_r0, sem_g0, sem_d0, sem_s0),
                (rows1, gv1, dst1, sem_r1, sem_g1, sem_d1, sem_s1))

        @pl.loop(0, CH)
        def _(i):
            @pl.loop(0, FG, step=16)
            def _(q):
                rows0[i, pl.ds(q, 16)] = jnp.zeros((16,), jnp.float32)

        @pl.loop(0, rows_per_sub, step=CH)
        def _(r):
            pltpu.sync_copy(rows0, acc.at[pl.ds(sid * rows_per_sub + r, CH)])

        pltpu.sync_copy(src_hbm.at[pl.ds(cb, n_ch)], src_all)

        plsc.subcore_barrier()

        def fire(cg, rows, gv, dstv, sem_r, sem_g, sem_d):
            pltpu.async_copy(lin_hbm.at[src_all.at[cg]], rows, sem_r)
            pltpu.async_copy(gate_hbm.at[pl.ds((cb + cg) * CH, CH)], gv, sem_g)
            pltpu.async_copy(dst_hbm.at[pl.ds(cb + cg, 1)], dstv, sem_d)

        for b in range(2):
            rows, gv, dstv, sem_r, sem_g, sem_d, _ = bufs[b]
            fire(b, rows, gv, dstv, sem_r, sem_g, sem_d)

        @pl.loop(0, n_ch, step=2)
        def _(c):
            for b in range(2):
                rows, gv, dstv, sem_r, sem_g, sem_d, sem_s = bufs[b]
                cg = c + b
                pltpu.make_async_copy(lin_hbm.at[src_all.at[cg]], rows,
                                      sem_r).wait()
                pltpu.make_async_copy(gate_hbm.at[pl.ds((cb + cg) * CH, CH)],
                                      gv, sem_g).wait()
                pltpu.make_async_copy(dst_hbm.at[pl.ds(cb + cg, 1)], dstv,
                                      sem_d).wait()

                @pl.loop(0, fv, step=16)
                def _(q, rows=rows, gv=gv):
                    @pl.loop(0, CH, step=4)
                    def _(i, q=q, rows=rows, gv=gv):
                        for u in range(4):
                            rows[i + u, pl.ds(q, 16)] = (
                                rows[i + u, pl.ds(q, 16)]
                                * gv[i + u, pl.ds(q, 16)])

                pltpu.async_copy(rows, acc.at[dstv.at[0]], sem_s, add=True)

            for b in range(2):
                rows, gv, dstv, sem_r, sem_g, sem_d, sem_s = bufs[b]
                cg = c + b

                @pl.when(cg + 2 < n_ch)
                def _(rows=rows, gv=gv, dstv=dstv, cg=cg,
                      sem_r=sem_r, sem_g=sem_g, sem_d=sem_d, sem_s=sem_s):
                    pltpu.make_async_copy(rows, acc.at[dst0.at[0]],
                                          sem_s).wait()
                    fire(cg + 2, rows, gv, dstv, sem_r, sem_g, sem_d)

        for b in range(2):
            rows = bufs[b][0]
            sem_s = bufs[b][6]
            pltpu.make_async_copy(rows, acc.at[dst0.at[0]], sem_s).wait()

        plsc.subcore_barrier()

        @pl.loop(0, rows_per_sub, step=CH)
        def _(r):
            row = sid * rows_per_sub + r
            pltpu.sync_copy(acc.at[pl.ds(row, CH)],
                            out_hbm.at[cid, pl.ds(row, CH)])

    return sc_kernel(lin, gate, src2, dst2)


def kernel(H, A, E, W1, b1, We1, be1, W2, b2, We2, be2):
    pad_e = EP - E_EDGES
    src = A[0]
    dst = A[1]
    srcp = jnp.concatenate([src, jnp.zeros((pad_e,), jnp.int32)])
    dstp = jnp.concatenate(
        [dst, N + (jnp.arange(pad_e, dtype=jnp.int32) % (NP - N))])
    src2 = srcp.reshape(EP // CH, CH)
    dst2 = dstp.reshape(EP // CH, CH)

    W1p = jnp.pad(W1, ((0, 0), (0, FG - 100)))
    b1p = jnp.pad(b1, (0, FG - 100)).reshape(1, FG)
    We1p = jnp.pad(We1, ((0, 0), (0, FG - 100)))
    be1p = jnp.pad(be1, (0, FG - 100)).reshape(1, FG)
    W2p = jnp.pad(W2, ((0, FG - 100), (0, FG - 40)))
    b2p = jnp.pad(b2, (0, FG - 40)).reshape(1, FG)
    We2p = jnp.pad(We2, ((0, 0), (0, FG - 40)))
    be2p = jnp.pad(be2, (0, FG - 40)).reshape(1, FG)

    lin1 = _node_linear(H, W1p, b1p)
    gate1 = _edge_gate(E, We1p, be1p)
    p1 = _sc_gather_mul_scatter(lin1, gate1, src2, dst2, F1)
    gate2 = _edge_gate(E, We2p, be2p)
    lin2 = _layer2_linear(p1[0], p1[1], lin1, W2p, b2p)
    p2 = _sc_gather_mul_scatter(lin2, gate2, src2, dst2, F2)
    return _final_logsoftmax(p2[0], p2[1], lin2)


# speedup vs baseline: 1.4859x; 1.3536x over previous
import functools

import jax
import jax.numpy as jnp
from jax import lax
from jax.experimental import pallas as pl
from jax.experimental.pallas import tpu as pltpu
from jax.experimental.pallas import tpu_sc as plsc

N = 10000
NP = 10240
E_EDGES = 160000
EP = 163840
D_EDGE = 16
FG = 128
F1 = 112
F2 = 48
NC, NS = 2, 16
NW = NC * NS
CH = 40
PT = EP // NW
MBLK = 1000
GBLK = 2048


def _lin_body(h_ref, w_ref, b_ref, o_ref):
    o_ref[...] = jnp.dot(h_ref[...], w_ref[...],
                         preferred_element_type=jnp.float32) + b_ref[...]


def _node_linear(h, w, b):
    m, k = h.shape
    f = w.shape[1]
    return pl.pallas_call(
        _lin_body,
        grid=(m // MBLK,),
        in_specs=[
            pl.BlockSpec((MBLK, k), lambda i: (i, 0)),
            pl.BlockSpec((k, f), lambda i: (0, 0)),
            pl.BlockSpec((1, f), lambda i: (0, 0)),
        ],
        out_specs=pl.BlockSpec((MBLK, f), lambda i: (i, 0)),
        out_shape=jax.ShapeDtypeStruct((m, f), jnp.float32),
    )(h, w, b)


def _gate_body(e_ref, w_ref, b_ref, o_ref):
    x = jnp.dot(e_ref[...].astype(jnp.bfloat16),
                w_ref[...].astype(jnp.bfloat16),
                preferred_element_type=jnp.float32) + b_ref[...]
    o_ref[...] = jax.nn.sigmoid(x)


def _edge_gate(e, w, b):
    f = w.shape[1]
    last_blk = E_EDGES // GBLK
    return pl.pallas_call(
        _gate_body,
        grid=(EP // GBLK,),
        in_specs=[
            pl.BlockSpec((GBLK, D_EDGE), lambda i: (jnp.minimum(i, last_blk), 0)),
            pl.BlockSpec((D_EDGE, f), lambda i: (0, 0)),
            pl.BlockSpec((1, f), lambda i: (0, 0)),
        ],
        out_specs=pl.BlockSpec((GBLK, f), lambda i: (i, 0)),
        out_shape=jax.ShapeDtypeStruct((EP, f), jnp.float32),
    )(e, w, b)


def _layer2_body(p0_ref, p1_ref, l_ref, w_ref, b_ref, o_ref):
    h = jnp.maximum(p0_ref[...] + p1_ref[...] + l_ref[...], 0.0)
    o_ref[...] = jnp.dot(h, w_ref[...],
                         preferred_element_type=jnp.float32) + b_ref[...]


def _layer2_linear(p0, p1, lin1, w, b):
    f = w.shape[1]
    return pl.pallas_call(
        _layer2_body,
        grid=(N // MBLK,),
        in_specs=[
            pl.BlockSpec((MBLK, FG), lambda i: (i, 0)),
            pl.BlockSpec((MBLK, FG), lambda i: (i, 0)),
            pl.BlockSpec((MBLK, FG), lambda i: (i, 0)),
            pl.BlockSpec((FG, f), lambda i: (0, 0)),
            pl.BlockSpec((1, f), lambda i: (0, 0)),
        ],
        out_specs=pl.BlockSpec((MBLK, f), lambda i: (i, 0)),
        out_shape=jax.ShapeDtypeStruct((N, f), jnp.float32),
    )(p0, p1, lin1, w, b)


def _final_body(p0_ref, p1_ref, l_ref, o_ref):
    x = p0_ref[...] + p1_ref[...] + l_ref[...]
    col = lax.broadcasted_iota(jnp.int32, x.shape, 1)
    xm = jnp.where(col < 40, x, -1e30)
    m = jnp.max(xm, axis=1, keepdims=True)
    lse = jnp.log(jnp.sum(jnp.exp(xm - m), axis=1, keepdims=True)) + m
    o_ref[...] = (x - lse)[:, :40]


def _final_logsoftmax(p0, p1, lin2):
    return pl.pallas_call(
        _final_body,
        grid=(N // MBLK,),
        in_specs=[
            pl.BlockSpec((MBLK, FG), lambda i: (i, 0)),
            pl.BlockSpec((MBLK, FG), lambda i: (i, 0)),
            pl.BlockSpec((MBLK, FG), lambda i: (i, 0)),
        ],
        out_specs=pl.BlockSpec((MBLK, 40), lambda i: (i, 0)),
        out_shape=jax.ShapeDtypeStruct((N, 40), jnp.float32),
    )(p0, p1, lin2)


def _sc_gather_mul_scatter(lin, gate, src2, dst2, fv):
    rows_per_sub = NP // NS
    n_ch = PT // CH
    mesh = plsc.VectorSubcoreMesh(core_axis_name="c", subcore_axis_name="s")

    @functools.partial(
        pl.kernel,
        out_type=jax.ShapeDtypeStruct((NC, NP, FG), jnp.float32),
        mesh=mesh,
        scratch_types=[
            pltpu.VMEM_SHARED((NP, FG), jnp.float32),
            pltpu.VMEM((n_ch, CH), jnp.int32),
            pltpu.VMEM((1, CH), jnp.int32),
            pltpu.VMEM((1, CH), jnp.int32),
            pltpu.VMEM((CH, FG), jnp.float32),
            pltpu.VMEM((CH, FG), jnp.float32),
            pltpu.VMEM((CH, FG), jnp.float32),
            pltpu.VMEM((CH, FG), jnp.float32),
            pltpu.SemaphoreType.DMA,
            pltpu.SemaphoreType.DMA,
            pltpu.SemaphoreType.DMA,
            pltpu.SemaphoreType.DMA,
            pltpu.SemaphoreType.DMA,
            pltpu.SemaphoreType.DMA,
            pltpu.SemaphoreType.DMA,
            pltpu.SemaphoreType.DMA,
        ],
    )
    def sc_kernel(lin_hbm, gate_hbm, src_hbm, dst_hbm, out_hbm,
                  acc, src_all, dst0, dst1, rows0, rows1, gv0, gv1,
                  sem_r0, sem_r1, sem_g0, sem_g1,
                  sem_d0, sem_d1, sem_s0, sem_s1):
        cid = lax.axis_index("c")
        sid = lax.axis_index("s")
        wid = sid * NC + cid
        cb = wid * n_ch

        bufs = ((rows0, gv0, dst0, sem_r0, sem_g0, sem_d0, sem_s0),
                (rows1, gv1, dst1, sem_r1, sem_g1, sem_d1, sem_s1))

        @pl.loop(0, CH)
        def _(i):
            @pl.loop(0, FG, step=16)
            def _(q):
                rows0[i, pl.ds(q, 16)] = jnp.zeros((16,), jnp.float32)

        @pl.loop(0, rows_per_sub, step=CH)
        def _(r):
            pltpu.sync_copy(rows0, acc.at[pl.ds(sid * rows_per_sub + r, CH)])

        pltpu.sync_copy(src_hbm.at[pl.ds(cb, n_ch)], src_all)

        plsc.subcore_barrier()

        def fire(cg, rows, gv, dstv, sem_r, sem_g, sem_d):
            pltpu.async_copy(lin_hbm.at[src_all.at[cg]], rows, sem_r)
            pltpu.async_copy(gate_hbm.at[pl.ds((cb + cg) * CH, CH)], gv, sem_g)
            pltpu.async_copy(dst_hbm.at[pl.ds(cb + cg, 1)], dstv, sem_d)

        for b in range(2):
            rows, gv, dstv, sem_r, sem_g, sem_d, _ = bufs[b]
            fire(b, rows, gv, dstv, sem_r, sem_g, sem_d)

        @pl.loop(0, n_ch, step=2)
        def _(c):
            for b in range(2):
                rows, gv, dstv, sem_r, sem_g, sem_d, sem_s = bufs[b]
                cg = c + b
                pltpu.make_async_copy(lin_hbm.at[src_all.at[cg]], rows,
                                      sem_r).wait()
                pltpu.make_async_copy(gate_hbm.at[pl.ds((cb + cg) * CH, CH)],
                                      gv, sem_g).wait()
                pltpu.make_async_copy(dst_hbm.at[pl.ds(cb + cg, 1)], dstv,
                                      sem_d).wait()

                @pl.loop(0, fv, step=16)
                def _(q, rows=rows, gv=gv):
                    @pl.loop(0, CH, step=4)
                    def _(i, q=q, rows=rows, gv=gv):
                        for u in range(4):
                            rows[i + u, pl.ds(q, 16)] = (
                                rows[i + u, pl.ds(q, 16)]
                                * gv[i + u, pl.ds(q, 16)])

                pltpu.async_copy(rows, acc.at[dstv.at[0]], sem_s, add=True)

            for b in range(2):
                rows, gv, dstv, sem_r, sem_g, sem_d, sem_s = bufs[b]
                cg = c + b

                @pl.when(cg + 2 < n_ch)
                def _(rows=rows, gv=gv, dstv=dstv, cg=cg,
                      sem_r=sem_r, sem_g=sem_g, sem_d=sem_d, sem_s=sem_s):
                    pltpu.make_async_copy(rows, acc.at[dst0.at[0]],
                                          sem_s).wait()
                    fire(cg + 2, rows, gv, dstv, sem_r, sem_g, sem_d)

        for b in range(2):
            rows = bufs[b][0]
            sem_s = bufs[b][6]
            pltpu.make_async_copy(rows, acc.at[dst0.at[0]], sem_s).wait()

        plsc.subcore_barrier()

        @pl.loop(0, rows_per_sub, step=CH)
        def _(r):
            row = sid * rows_per_sub + r
            pltpu.sync_copy(acc.at[pl.ds(row, CH)],
                            out_hbm.at[cid, pl.ds(row, CH)])

    return sc_kernel(lin, gate, src2, dst2)


def kernel(H, A, E, W1, b1, We1, be1, W2, b2, We2, be2):
    pad_e = EP - E_EDGES
    src = A[0]
    dst = A[1]
    srcp = jnp.concatenate(
        [src, jnp.arange(pad_e, dtype=jnp.int32) % N])
    dstp = jnp.concatenate(
        [dst, N + (jnp.arange(pad_e, dtype=jnp.int32) % (NP - N))])
    src2 = srcp.reshape(EP // CH, CH)
    dst2 = dstp.reshape(EP // CH, CH)

    W1p = jnp.pad(W1, ((0, 0), (0, FG - 100)))
    b1p = jnp.pad(b1, (0, FG - 100)).reshape(1, FG)
    We1p = jnp.pad(We1, ((0, 0), (0, FG - 100)))
    be1p = jnp.pad(be1, (0, FG - 100)).reshape(1, FG)
    W2p = jnp.pad(W2, ((0, FG - 100), (0, FG - 40)))
    b2p = jnp.pad(b2, (0, FG - 40)).reshape(1, FG)
    We2p = jnp.pad(We2, ((0, 0), (0, FG - 40)))
    be2p = jnp.pad(be2, (0, FG - 40)).reshape(1, FG)

    lin1 = _node_linear(H, W1p, b1p)
    gate1 = _edge_gate(E, We1p, be1p)
    p1 = _sc_gather_mul_scatter(lin1, gate1, src2, dst2, F1)
    gate2 = _edge_gate(E, We2p, be2p)
    lin2 = _layer2_linear(p1[0], p1[1], lin1, W2p, b2p)
    p2 = _sc_gather_mul_scatter(lin2, gate2, src2, dst2, F2)
    return _final_logsoftmax(p2[0], p2[1], lin2)


# speedup vs baseline: 1.9152x; 1.2889x over previous
import functools

import jax
import jax.numpy as jnp
from jax import lax
from jax.experimental import pallas as pl
from jax.experimental.pallas import tpu as pltpu
from jax.experimental.pallas import tpu_sc as plsc

N = 10000
NP = 10240
E_EDGES = 160000
EP = 163840
D_EDGE = 16
FG = 128
F1 = 112
F2 = 48
NC, NS = 2, 16
NW = NC * NS
CH = 40
PT = EP // NW
MBLK = 1000
GBLK = 2048


def _lin_body(h_ref, w_ref, b_ref, o_ref):
    o_ref[...] = jnp.dot(h_ref[...], w_ref[...],
                         preferred_element_type=jnp.float32) + b_ref[...]


def _node_linear(h, w, b):
    m, k = h.shape
    f = w.shape[1]
    return pl.pallas_call(
        _lin_body,
        grid=(m // MBLK,),
        in_specs=[
            pl.BlockSpec((MBLK, k), lambda i: (i, 0)),
            pl.BlockSpec((k, f), lambda i: (0, 0)),
            pl.BlockSpec((1, f), lambda i: (0, 0)),
        ],
        out_specs=pl.BlockSpec((MBLK, f), lambda i: (i, 0)),
        out_shape=jax.ShapeDtypeStruct((m, f), jnp.float32),
    )(h, w, b)


def _gate_body(e_ref, w_ref, b_ref, o_ref):
    x = jnp.dot(e_ref[...].astype(jnp.bfloat16),
                w_ref[...].astype(jnp.bfloat16),
                preferred_element_type=jnp.float32) + b_ref[...]
    o_ref[...] = jax.nn.sigmoid(x)


def _edge_gate(e, w, b):
    f = w.shape[1]
    last_blk = E_EDGES // GBLK
    return pl.pallas_call(
        _gate_body,
        grid=(EP // GBLK,),
        in_specs=[
            pl.BlockSpec((GBLK, D_EDGE), lambda i: (jnp.minimum(i, last_blk), 0)),
            pl.BlockSpec((D_EDGE, f), lambda i: (0, 0)),
            pl.BlockSpec((1, f), lambda i: (0, 0)),
        ],
        out_specs=pl.BlockSpec((GBLK, f), lambda i: (i, 0)),
        out_shape=jax.ShapeDtypeStruct((EP, f), jnp.float32),
    )(e, w, b)


def _layer2_body(p0_ref, p1_ref, l_ref, w_ref, b_ref, o_ref):
    h = jnp.maximum(p0_ref[...] + p1_ref[...] + l_ref[...], 0.0)
    o_ref[...] = jnp.dot(h, w_ref[...],
                         preferred_element_type=jnp.float32) + b_ref[...]


def _layer2_linear(p0, p1, lin1, w, b):
    f = w.shape[1]
    return pl.pallas_call(
        _layer2_body,
        grid=(N // MBLK,),
        in_specs=[
            pl.BlockSpec((MBLK, FG), lambda i: (i, 0)),
            pl.BlockSpec((MBLK, FG), lambda i: (i, 0)),
            pl.BlockSpec((MBLK, FG), lambda i: (i, 0)),
            pl.BlockSpec((FG, f), lambda i: (0, 0)),
            pl.BlockSpec((1, f), lambda i: (0, 0)),
        ],
        out_specs=pl.BlockSpec((MBLK, f), lambda i: (i, 0)),
        out_shape=jax.ShapeDtypeStruct((N, f), jnp.float32),
    )(p0, p1, lin1, w, b)


def _final_body(p0_ref, p1_ref, l_ref, o_ref):
    x = p0_ref[...] + p1_ref[...] + l_ref[...]
    col = lax.broadcasted_iota(jnp.int32, x.shape, 1)
    xm = jnp.where(col < 40, x, -1e30)
    m = jnp.max(xm, axis=1, keepdims=True)
    lse = jnp.log(jnp.sum(jnp.exp(xm - m), axis=1, keepdims=True)) + m
    o_ref[...] = (x - lse)[:, :40]


def _final_logsoftmax(p0, p1, lin2):
    return pl.pallas_call(
        _final_body,
        grid=(N // MBLK,),
        in_specs=[
            pl.BlockSpec((MBLK, FG), lambda i: (i, 0)),
            pl.BlockSpec((MBLK, FG), lambda i: (i, 0)),
            pl.BlockSpec((MBLK, FG), lambda i: (i, 0)),
        ],
        out_specs=pl.BlockSpec((MBLK, 40), lambda i: (i, 0)),
        out_shape=jax.ShapeDtypeStruct((N, 40), jnp.float32),
    )(p0, p1, lin2)


def _sc_gather_mul_scatter(lin, gate, src2, dst2, fv):
    rows_per_sub = NP // NS
    n_ch = PT // CH
    mesh = plsc.VectorSubcoreMesh(core_axis_name="c", subcore_axis_name="s")

    @functools.partial(
        pl.kernel,
        out_type=jax.ShapeDtypeStruct((NC, NP, FG), jnp.float32),
        mesh=mesh,
        scratch_types=[
            pltpu.VMEM_SHARED((NP, FG), jnp.float32),
            pltpu.VMEM((PT,), jnp.int32),
            pltpu.VMEM((CH,), jnp.int32),
            pltpu.VMEM((CH,), jnp.int32),
            pltpu.VMEM((CH, FG), jnp.float32),
            pltpu.VMEM((CH, FG), jnp.float32),
            pltpu.VMEM((CH, FG), jnp.float32),
            pltpu.VMEM((CH, FG), jnp.float32),
            pltpu.SemaphoreType.DMA,
            pltpu.SemaphoreType.DMA,
            pltpu.SemaphoreType.DMA,
            pltpu.SemaphoreType.DMA,
            pltpu.SemaphoreType.DMA,
            pltpu.SemaphoreType.DMA,
            pltpu.SemaphoreType.DMA,
            pltpu.SemaphoreType.DMA,
        ],
    )
    def sc_kernel(lin_hbm, gate_hbm, src_hbm, dst_hbm, out_hbm,
                  acc, src_all, dst0, dst1, rows0, rows1, gv0, gv1,
                  sem_r0, sem_r1, sem_g0, sem_g1,
                  sem_d0, sem_d1, sem_s0, sem_s1):
        cid = lax.axis_index("c")
        sid = lax.axis_index("s")
        wid = sid * NC + cid
        cb = wid * n_ch

        bufs = ((rows0, gv0, dst0, sem_r0, sem_g0, sem_d0, sem_s0),
                (rows1, gv1, dst1, sem_r1, sem_g1, sem_d1, sem_s1))

        @pl.loop(0, CH)
        def _(i):
            @pl.loop(0, FG, step=16)
            def _(q):
                rows0[i, pl.ds(q, 16)] = jnp.zeros((16,), jnp.float32)

        @pl.loop(0, rows_per_sub, step=CH)
        def _(r):
            pltpu.sync_copy(rows0, acc.at[pl.ds(sid * rows_per_sub + r, CH)])

        pltpu.sync_copy(src_hbm.at[pl.ds(wid * PT, PT)], src_all)

        plsc.subcore_barrier()

        def fire(cg, rows, gv, dstv, sem_r, sem_g, sem_d):
            pltpu.async_copy(lin_hbm.at[src_all.at[pl.ds(cg * CH, CH)]],
                             rows, sem_r)
            pltpu.async_copy(gate_hbm.at[pl.ds((cb + cg) * CH, CH)], gv, sem_g)
            pltpu.async_copy(dst_hbm.at[pl.ds(wid * PT + cg * CH, CH)],
                             dstv, sem_d)

        for b in range(2):
            rows, gv, dstv, sem_r, sem_g, sem_d, _ = bufs[b]
            fire(b, rows, gv, dstv, sem_r, sem_g, sem_d)

        @pl.loop(0, n_ch, step=2)
        def _(c):
            for b in range(2):
                rows, gv, dstv, sem_r, sem_g, sem_d, sem_s = bufs[b]
                cg = c + b
                pltpu.make_async_copy(lin_hbm.at[src_all.at[pl.ds(cg * CH, CH)]],
                                      rows, sem_r).wait()
                pltpu.make_async_copy(gate_hbm.at[pl.ds((cb + cg) * CH, CH)],
                                      gv, sem_g).wait()
                pltpu.make_async_copy(dst_hbm.at[pl.ds(wid * PT + cg * CH, CH)],
                                      dstv, sem_d).wait()

                @pl.loop(0, CH)
                def _(i, rows=rows, gv=gv):
                    for q in range(0, fv, 16):
                        rows[i, pl.ds(q, 16)] = (rows[i, pl.ds(q, 16)]
                                                 * gv[i, pl.ds(q, 16)])

                pltpu.async_copy(rows, acc.at[dstv], sem_s, add=True)

            for b in range(2):
                rows, gv, dstv, sem_r, sem_g, sem_d, sem_s = bufs[b]
                cg = c + b

                @pl.when(cg + 2 < n_ch)
                def _(rows=rows, gv=gv, dstv=dstv, cg=cg,
                      sem_r=sem_r, sem_g=sem_g, sem_d=sem_d, sem_s=sem_s):
                    pltpu.make_async_copy(rows, acc.at[dst0],
                                          sem_s).wait()
                    fire(cg + 2, rows, gv, dstv, sem_r, sem_g, sem_d)

        for b in range(2):
            rows = bufs[b][0]
            sem_s = bufs[b][6]
            pltpu.make_async_copy(rows, acc.at[dst0], sem_s).wait()

        plsc.subcore_barrier()

        @pl.loop(0, rows_per_sub, step=CH)
        def _(r):
            row = sid * rows_per_sub + r
            pltpu.sync_copy(acc.at[pl.ds(row, CH)],
                            out_hbm.at[cid, pl.ds(row, CH)])

    return sc_kernel(lin, gate, src2, dst2)


def kernel(H, A, E, W1, b1, We1, be1, W2, b2, We2, be2):
    pad_e = EP - E_EDGES
    src = A[0]
    dst = A[1]
    srcp = jnp.concatenate(
        [src, jnp.arange(pad_e, dtype=jnp.int32) % N])
    dstp = jnp.concatenate(
        [dst, N + (jnp.arange(pad_e, dtype=jnp.int32) % (NP - N))])

    W1p = jnp.pad(W1, ((0, 0), (0, FG - 100)))
    b1p = jnp.pad(b1, (0, FG - 100)).reshape(1, FG)
    We1p = jnp.pad(We1, ((0, 0), (0, FG - 100)))
    be1p = jnp.pad(be1, (0, FG - 100)).reshape(1, FG)
    W2p = jnp.pad(W2, ((0, FG - 100), (0, FG - 40)))
    b2p = jnp.pad(b2, (0, FG - 40)).reshape(1, FG)
    We2p = jnp.pad(We2, ((0, 0), (0, FG - 40)))
    be2p = jnp.pad(be2, (0, FG - 40)).reshape(1, FG)

    lin1 = _node_linear(H, W1p, b1p)
    gate1 = _edge_gate(E, We1p, be1p)
    p1 = _sc_gather_mul_scatter(lin1, gate1, srcp, dstp, F1)
    gate2 = _edge_gate(E, We2p, be2p)
    lin2 = _layer2_linear(p1[0], p1[1], lin1, W2p, b2p)
    p2 = _sc_gather_mul_scatter(lin2, gate2, srcp, dstp, F2)
    return _final_logsoftmax(p2[0], p2[1], lin2)
